# baseline (speedup 1.0000x reference)
import functools

import jax
import jax.numpy as jnp
from jax import lax
from jax.experimental import pallas as pl
from jax.experimental.pallas import tpu as pltpu
from jax.experimental.pallas import tpu_sc as plsc

_NC = 2
_NS = 16
_NW = _NC * _NS

_XR = 4096
_XC = 16
_B = _XR * _XC
_D = 768
_BPW = _B // _NW
_GPW = _BPW // _XC


@functools.partial(
    pl.kernel,
    out_type=jax.ShapeDtypeStruct((_B, _D), jnp.float32),
    mesh=plsc.VectorSubcoreMesh(core_axis_name="c", subcore_axis_name="s"),
    scratch_types=[
        pltpu.VMEM((_GPW, _XC), jnp.int32),
        pltpu.VMEM((16, _D), jnp.float32),
        pltpu.SemaphoreType.DMA,
    ],
)
def _gather_rows(x_hbm, table_hbm, out_hbm, idx_v, table_v, sem):
    wid = lax.axis_index("s") * _NC + lax.axis_index("c")
    base = wid * _BPW
    pltpu.sync_copy(table_hbm, table_v)
    pltpu.sync_copy(x_hbm.at[pl.ds(wid * _GPW, _GPW)], idx_v)

    def group(g, _):
        vidx = idx_v[g]
        for r in range(_XC):
            i = vidx[r]
            pltpu.async_copy(table_v.at[i], out_hbm.at[base + g * _XC + r], sem)
        return ()

    lax.fori_loop(0, _GPW, group, ())

    def drain(g, _):
        pltpu.make_async_copy(
            table_v, out_hbm.at[pl.ds(base + g * 16, 16)], sem
        ).wait()
        return ()

    lax.fori_loop(0, _GPW, drain, ())


def kernel(x, table):
    out = _gather_rows(x.astype(jnp.int32), table)
    return out.reshape(x.shape + (table.shape[1],))
